# baseline (speedup 1.0000x reference)
import functools

import jax
import jax.numpy as jnp
from jax.experimental import pallas as pl
from jax.experimental.pallas import tpu as pltpu

HIDDEN = 768
INTER = 2048
E = 64
TOP_K = 1
CAP_FACTOR = 1.25
AUX_ALPHA = 0.01
Z_ALPHA = 0.001


def _router_kernel(x_ref, wr_ref, lt_ref, idx_ref, w_ref, recip_ref, *, cap):
    wr = wr_ref[...]
    S = x_ref.shape[0]
    lt = jax.lax.dot_general(wr, x_ref[...], (((1,), (1,)), ((), ())),
                             preferred_element_type=jnp.float32)
    lt_ref[...] = lt

    iota = jax.lax.broadcasted_iota(jnp.int32, (E, S), 1)
    iota_cap = jax.lax.broadcasted_iota(jnp.int32, (E, cap), 1)
    neg_inf = jnp.float32(-jnp.inf)

    def body(k, carry):
        cur, idxs, vals = carry
        m = jnp.max(cur, axis=1, keepdims=True)
        first = jnp.min(jnp.where(cur == m, iota, S), axis=1)
        sel_col = iota_cap == k
        idxs = jnp.where(sel_col, first[:, None], idxs)
        vals = jnp.where(sel_col, m, vals)
        cur = jnp.where(iota == first[:, None], neg_inf, cur)
        return cur, idxs, vals

    idxs0 = jnp.zeros((E, cap), dtype=jnp.int32)
    vals0 = jnp.zeros((E, cap), dtype=jnp.float32)
    cur, idxs, vals = jax.lax.fori_loop(0, cap, body, (lt, idxs0, vals0),
                                        unroll=8)

    idx_ref[...] = idxs
    w_ref[...] = jax.nn.softmax(vals, axis=-1)
    counts = jnp.sum(jnp.where(cur == neg_inf, 1.0, 0.0), axis=0)
    recip_ref[...] = (1.0 / jnp.maximum(counts, 1.0))[None, :]


def _moe_kernel(idx_ref, x_ref, w_ref, w1_ref, w2_ref, recip_ref, lt_ref,
                out_ref, loss_ref, sel_ref, o_ref, *, cap, n_experts):
    e = pl.program_id(0)
    c = pl.program_id(1)

    @pl.when((e == 0) & (c == 0))
    def _init():
        out_ref[...] = jnp.zeros_like(out_ref)

    @pl.when(c == 0)
    def _gather():
        def gather(i, _):
            t = idx_ref[e * cap + i]
            sel_ref[pl.ds(i, 1), :] = x_ref[pl.ds(t, 1), :]
            return 0

        jax.lax.fori_loop(0, cap, gather, 0, unroll=True)

    sel = sel_ref[...]
    h = jax.lax.dot_general(sel, w1_ref[0], (((1,), (1,)), ((), ())),
                            preferred_element_type=jnp.float32)
    h = h * jax.nn.sigmoid(h)
    o = jax.lax.dot_general(h, w2_ref[0], (((1,), (1,)), ((), ())),
                            preferred_element_type=jnp.float32)

    @pl.when(c == 0)
    def _store_first():
        o_ref[...] = o

    @pl.when(c == 1)
    def _scatter():
        o_ref[...] = (o_ref[...] + o) * w_ref[0]

        def scatter(i, _):
            t = idx_ref[e * cap + i]
            out_ref[pl.ds(t, 1), :] += o_ref[pl.ds(i, 1), :]
            return 0

        jax.lax.fori_loop(0, cap, scatter, 0, unroll=True)

    @pl.when((e == n_experts - 1) & (c == 1))
    def _finish():
        out_ref[...] = out_ref[...] * recip_ref[...]
        lt = lt_ref[...]
        S = lt.shape[1]
        mx = jnp.max(lt, axis=0, keepdims=True)
        ex = jnp.exp(lt - mx)
        probs = ex / jnp.sum(ex, axis=0, keepdims=True)
        usage = jnp.mean(probs, axis=1, keepdims=True)
        aux = jnp.mean((usage - 1.0 / n_experts) ** 2) * n_experts
        z = jnp.mean(lt * lt)
        loss_ref[...] = jnp.broadcast_to(AUX_ALPHA * aux + Z_ALPHA * z, (1, 1))


@jax.jit
def kernel(x, Wr, W1, W2):
    Bn, S, H = x.shape
    cap = int(min(S, max(1, S * TOP_K // E * CAP_FACTOR)))
    x2 = x.reshape(Bn * S, H)
    Sf = Bn * S

    lt, idxs, weights, recip = pl.pallas_call(
        functools.partial(_router_kernel, cap=cap),
        out_shape=(
            jax.ShapeDtypeStruct((E, Sf), jnp.float32),
            jax.ShapeDtypeStruct((E, cap), jnp.int32),
            jax.ShapeDtypeStruct((E, cap), jnp.float32),
            jax.ShapeDtypeStruct((1, Sf), jnp.float32),
        ),
    )(x2, Wr)

    idx_flat = idxs.reshape(E * cap)
    w3 = weights.reshape(E, cap, 1)
    recip_col = recip.reshape(Sf, 1)

    grid_spec = pltpu.PrefetchScalarGridSpec(
        num_scalar_prefetch=1,
        grid=(E, 2),
        in_specs=[
            pl.BlockSpec((Sf, H), lambda e, c, idx: (0, 0)),
            pl.BlockSpec((1, cap, 1), lambda e, c, idx: (e, 0, 0)),
            pl.BlockSpec((1, INTER // 2, H), lambda e, c, idx: (e, c, 0)),
            pl.BlockSpec((1, H, INTER // 2), lambda e, c, idx: (e, 0, c)),
            pl.BlockSpec((Sf, 1), lambda e, c, idx: (0, 0)),
            pl.BlockSpec((E, Sf), lambda e, c, idx: (0, 0)),
        ],
        out_specs=[
            pl.BlockSpec((Sf, H), lambda e, c, idx: (0, 0)),
            pl.BlockSpec((1, 1), lambda e, c, idx: (0, 0)),
        ],
        scratch_shapes=[
            pltpu.VMEM((cap, H), jnp.float32),
            pltpu.VMEM((cap, H), jnp.float32),
        ],
    )

    out, loss = pl.pallas_call(
        functools.partial(_moe_kernel, cap=cap, n_experts=E),
        grid_spec=grid_spec,
        out_shape=(
            jax.ShapeDtypeStruct((Sf, H), jnp.float32),
            jax.ShapeDtypeStruct((1, 1), jnp.float32),
        ),
        compiler_params=pltpu.CompilerParams(
            dimension_semantics=("arbitrary", "arbitrary"),
        ),
    )(idx_flat, x2, w3, W1, W2, recip_col, lt)

    return out.reshape(Bn, S, H), loss[0, 0]


# speedup vs baseline: 1.1923x; 1.1923x over previous
import functools

import jax
import jax.numpy as jnp
from jax.experimental import pallas as pl
from jax.experimental.pallas import tpu as pltpu

HIDDEN = 768
INTER = 2048
E = 64
TOP_K = 1
CAP_FACTOR = 1.25
AUX_ALPHA = 0.01
Z_ALPHA = 0.001


def _router_kernel(x_ref, wr_ref, lt_ref, idx_ref, w_ref, recip_ref, *, cap):
    wr = wr_ref[...]
    S = x_ref.shape[0]
    lt = jax.lax.dot_general(wr, x_ref[...], (((1,), (1,)), ((), ())),
                             preferred_element_type=jnp.float32)
    lt_ref[...] = lt

    iota = jax.lax.broadcasted_iota(jnp.int32, (E, S), 1)
    iota_cap = jax.lax.broadcasted_iota(jnp.int32, (E, cap), 1)
    neg_inf = jnp.float32(-jnp.inf)

    def body(k, carry):
        cur, idxs, vals = carry
        m = jnp.max(cur, axis=1, keepdims=True)
        first = jnp.min(jnp.where(cur == m, iota, S), axis=1)
        sel_col = iota_cap == k
        idxs = jnp.where(sel_col, first[:, None], idxs)
        vals = jnp.where(sel_col, m, vals)
        cur = jnp.where(iota == first[:, None], neg_inf, cur)
        return cur, idxs, vals

    idxs0 = jnp.zeros((E, cap), dtype=jnp.int32)
    vals0 = jnp.zeros((E, cap), dtype=jnp.float32)
    cur, idxs, vals = jax.lax.fori_loop(0, cap, body, (lt, idxs0, vals0),
                                        unroll=16)

    idx_ref[...] = idxs
    w_ref[...] = jax.nn.softmax(vals, axis=-1)
    counts = jnp.sum(jnp.where(cur == neg_inf, 1.0, 0.0), axis=0)
    recip_ref[...] = (1.0 / jnp.maximum(counts, 1.0))[None, :]


def _moe_kernel(idx_ref, x_ref, w_ref, w1_ref, w2_ref, recip_ref, lt_ref,
                out_ref, loss_ref, sel_ref, o_ref, *, cap, n_experts):
    e = pl.program_id(0)

    @pl.when(e == 0)
    def _init():
        out_ref[...] = jnp.zeros_like(out_ref)

    def gather(i, _):
        t = idx_ref[e * cap + i]
        sel_ref[pl.ds(i, 1), :] = x_ref[pl.ds(t, 1), :]
        return 0

    jax.lax.fori_loop(0, cap, gather, 0, unroll=True)

    sel = sel_ref[...]
    h = jax.lax.dot_general(sel, w1_ref[0], (((1,), (1,)), ((), ())),
                            preferred_element_type=jnp.float32)
    h = h * jax.nn.sigmoid(h)
    o = jax.lax.dot_general(h, w2_ref[0], (((1,), (1,)), ((), ())),
                            preferred_element_type=jnp.float32)
    o_ref[...] = o * w_ref[0]

    def scatter(i, _):
        t = idx_ref[e * cap + i]
        out_ref[pl.ds(t, 1), :] += o_ref[pl.ds(i, 1), :]
        return 0

    jax.lax.fori_loop(0, cap, scatter, 0, unroll=True)

    @pl.when(e == n_experts - 1)
    def _finish():
        out_ref[...] = out_ref[...] * recip_ref[...]
        lt = lt_ref[...]
        S = lt.shape[1]
        mx = jnp.max(lt, axis=0, keepdims=True)
        ex = jnp.exp(lt - mx)
        probs = ex / jnp.sum(ex, axis=0, keepdims=True)
        usage = jnp.mean(probs, axis=1, keepdims=True)
        aux = jnp.mean((usage - 1.0 / n_experts) ** 2) * n_experts
        z = jnp.mean(lt * lt)
        loss_ref[...] = jnp.broadcast_to(AUX_ALPHA * aux + Z_ALPHA * z, (1, 1))


@jax.jit
def kernel(x, Wr, W1, W2):
    Bn, S, H = x.shape
    cap = int(min(S, max(1, S * TOP_K // E * CAP_FACTOR)))
    x2 = x.reshape(Bn * S, H)
    Sf = Bn * S

    lt, idxs, weights, recip = pl.pallas_call(
        functools.partial(_router_kernel, cap=cap),
        out_shape=(
            jax.ShapeDtypeStruct((E, Sf), jnp.float32),
            jax.ShapeDtypeStruct((E, cap), jnp.int32),
            jax.ShapeDtypeStruct((E, cap), jnp.float32),
            jax.ShapeDtypeStruct((1, Sf), jnp.float32),
        ),
    )(x2, Wr)

    idx_flat = idxs.reshape(E * cap)
    w3 = weights.reshape(E, cap, 1)
    recip_col = recip.reshape(Sf, 1)

    grid_spec = pltpu.PrefetchScalarGridSpec(
        num_scalar_prefetch=1,
        grid=(E,),
        in_specs=[
            pl.BlockSpec((Sf, H), lambda e, idx: (0, 0)),
            pl.BlockSpec((1, cap, 1), lambda e, idx: (e, 0, 0)),
            pl.BlockSpec((1, INTER, H), lambda e, idx: (e, 0, 0)),
            pl.BlockSpec((1, H, INTER), lambda e, idx: (e, 0, 0)),
            pl.BlockSpec((Sf, 1), lambda e, idx: (0, 0)),
            pl.BlockSpec((E, Sf), lambda e, idx: (0, 0)),
        ],
        out_specs=[
            pl.BlockSpec((Sf, H), lambda e, idx: (0, 0)),
            pl.BlockSpec((1, 1), lambda e, idx: (0, 0)),
        ],
        scratch_shapes=[
            pltpu.VMEM((cap, H), jnp.float32),
            pltpu.VMEM((cap, H), jnp.float32),
        ],
    )

    out, loss = pl.pallas_call(
        functools.partial(_moe_kernel, cap=cap, n_experts=E),
        grid_spec=grid_spec,
        out_shape=(
            jax.ShapeDtypeStruct((Sf, H), jnp.float32),
            jax.ShapeDtypeStruct((1, 1), jnp.float32),
        ),
        compiler_params=pltpu.CompilerParams(
            dimension_semantics=("arbitrary",),
        ),
    )(idx_flat, x2, w3, W1, W2, recip_col, lt)

    return out.reshape(Bn, S, H), loss[0, 0]


# speedup vs baseline: 1.1949x; 1.0021x over previous
import functools

import jax
import jax.numpy as jnp
from jax.experimental import pallas as pl
from jax.experimental.pallas import tpu as pltpu

HIDDEN = 768
INTER = 2048
E = 64
TOP_K = 1
CAP_FACTOR = 1.25
AUX_ALPHA = 0.01
Z_ALPHA = 0.001


def _router_kernel(x_ref, wr_ref, lt_ref, idx_ref, w_ref, recip_ref, *, cap):
    wr = wr_ref[...]
    S = x_ref.shape[0]
    lt = jax.lax.dot_general(wr, x_ref[...], (((1,), (1,)), ((), ())),
                             preferred_element_type=jnp.float32)
    lt_ref[...] = lt

    iota = jax.lax.broadcasted_iota(jnp.int32, (E, S), 1)
    iota_cap = jax.lax.broadcasted_iota(jnp.int32, (E, cap), 1)
    neg_inf = jnp.float32(-jnp.inf)

    def body(k, carry):
        cur, idxs, vals = carry
        m = jnp.max(cur, axis=1, keepdims=True)
        first = jnp.min(jnp.where(cur == m, iota, S), axis=1)
        sel_col = iota_cap == k
        idxs = jnp.where(sel_col, first[:, None], idxs)
        vals = jnp.where(sel_col, m, vals)
        cur = jnp.where(iota == first[:, None], neg_inf, cur)
        return cur, idxs, vals

    idxs0 = jnp.zeros((E, cap), dtype=jnp.int32)
    vals0 = jnp.zeros((E, cap), dtype=jnp.float32)
    cur, idxs, vals = jax.lax.fori_loop(0, cap, body, (lt, idxs0, vals0),
                                        unroll=40)

    idx_ref[...] = idxs
    w_ref[...] = jax.nn.softmax(vals, axis=-1)
    counts = jnp.sum(jnp.where(cur == neg_inf, 1.0, 0.0), axis=0)
    recip_ref[...] = (1.0 / jnp.maximum(counts, 1.0))[None, :]


def _moe_kernel(idx_ref, x_ref, w_ref, w1_ref, w2_ref, recip_ref, lt_ref,
                out_ref, loss_ref, sel_ref, o_ref, *, cap, n_experts):
    e = pl.program_id(0)

    @pl.when(e == 0)
    def _init():
        out_ref[...] = jnp.zeros_like(out_ref)

    def gather(i, _):
        t = idx_ref[e * cap + i]
        sel_ref[pl.ds(i, 1), :] = x_ref[pl.ds(t, 1), :]
        return 0

    jax.lax.fori_loop(0, cap, gather, 0, unroll=True)

    sel = sel_ref[...]
    h = jax.lax.dot_general(sel, w1_ref[0], (((1,), (1,)), ((), ())),
                            preferred_element_type=jnp.float32)
    h = h * jax.nn.sigmoid(h)
    o = jax.lax.dot_general(h, w2_ref[0], (((1,), (1,)), ((), ())),
                            preferred_element_type=jnp.float32)
    o_ref[...] = o * w_ref[0]

    def scatter(i, _):
        t = idx_ref[e * cap + i]
        out_ref[pl.ds(t, 1), :] += o_ref[pl.ds(i, 1), :]
        return 0

    jax.lax.fori_loop(0, cap, scatter, 0, unroll=True)

    @pl.when(e == n_experts - 1)
    def _finish():
        out_ref[...] = out_ref[...] * recip_ref[...]
        lt = lt_ref[...]
        S = lt.shape[1]
        mx = jnp.max(lt, axis=0, keepdims=True)
        ex = jnp.exp(lt - mx)
        probs = ex / jnp.sum(ex, axis=0, keepdims=True)
        usage = jnp.mean(probs, axis=1, keepdims=True)
        aux = jnp.mean((usage - 1.0 / n_experts) ** 2) * n_experts
        z = jnp.mean(lt * lt)
        loss_ref[...] = jnp.broadcast_to(AUX_ALPHA * aux + Z_ALPHA * z, (1, 1))


@jax.jit
def kernel(x, Wr, W1, W2):
    Bn, S, H = x.shape
    cap = int(min(S, max(1, S * TOP_K // E * CAP_FACTOR)))
    x2 = x.reshape(Bn * S, H)
    Sf = Bn * S

    lt, idxs, weights, recip = pl.pallas_call(
        functools.partial(_router_kernel, cap=cap),
        out_shape=(
            jax.ShapeDtypeStruct((E, Sf), jnp.float32),
            jax.ShapeDtypeStruct((E, cap), jnp.int32),
            jax.ShapeDtypeStruct((E, cap), jnp.float32),
            jax.ShapeDtypeStruct((1, Sf), jnp.float32),
        ),
    )(x2, Wr)

    idx_flat = idxs.reshape(E * cap)
    w3 = weights.reshape(E, cap, 1)
    recip_col = recip.reshape(Sf, 1)

    grid_spec = pltpu.PrefetchScalarGridSpec(
        num_scalar_prefetch=1,
        grid=(E,),
        in_specs=[
            pl.BlockSpec((Sf, H), lambda e, idx: (0, 0)),
            pl.BlockSpec((1, cap, 1), lambda e, idx: (e, 0, 0)),
            pl.BlockSpec((1, INTER, H), lambda e, idx: (e, 0, 0)),
            pl.BlockSpec((1, H, INTER), lambda e, idx: (e, 0, 0)),
            pl.BlockSpec((Sf, 1), lambda e, idx: (0, 0)),
            pl.BlockSpec((E, Sf), lambda e, idx: (0, 0)),
        ],
        out_specs=[
            pl.BlockSpec((Sf, H), lambda e, idx: (0, 0)),
            pl.BlockSpec((1, 1), lambda e, idx: (0, 0)),
        ],
        scratch_shapes=[
            pltpu.VMEM((cap, H), jnp.float32),
            pltpu.VMEM((cap, H), jnp.float32),
        ],
    )

    out, loss = pl.pallas_call(
        functools.partial(_moe_kernel, cap=cap, n_experts=E),
        grid_spec=grid_spec,
        out_shape=(
            jax.ShapeDtypeStruct((Sf, H), jnp.float32),
            jax.ShapeDtypeStruct((1, 1), jnp.float32),
        ),
        compiler_params=pltpu.CompilerParams(
            dimension_semantics=("arbitrary",),
        ),
    )(idx_flat, x2, w3, W1, W2, recip_col, lt)

    return out.reshape(Bn, S, H), loss[0, 0]
